# baseline (speedup 1.0000x reference)
import dataclasses

import jax
import jax.numpy as jnp
from jax import lax
from jax.experimental import pallas as pl
from jax.experimental.pallas import tpu as pltpu
from jax.experimental.pallas import tpu_sc as plsc

N = 10000
E = 320000
IN_DIM = 128
D = 128
ALPHA = 0.2

NC = 2
NS = 16
NW = NC * NS
BLK = 128
CHK = 16
NCHUNK = 5
EPW = NCHUNK * CHK * BLK
EPAD = NW * EPW
NPAD = 10240
RPT = NPAD // NS
DLN = 16
LANES = 16
NEG = -1e30


def _prologue_body(feat_ref, w_ref, al_ref, ar_ref, ft_ref, el_ref, er_ref,
                   m_ref):
    ft = jnp.dot(feat_ref[...], w_ref[...], preferred_element_type=jnp.float32)
    ft_ref[:N] = ft
    ft_ref[N:] = jnp.zeros((NPAD - N, D), jnp.float32)
    el = jnp.sum(ft * al_ref[...], axis=1, keepdims=True)
    er = jnp.sum(ft * ar_ref[...], axis=1, keepdims=True)
    el_ref[:N] = el
    el_ref[N:] = jnp.full((NPAD - N, 1), NEG, jnp.float32)
    er_ref[:N] = er
    er_ref[N:] = jnp.full((NPAD - N, 1), NEG, jnp.float32)
    s = jnp.max(el) + jnp.max(er)
    m_ref[...] = jnp.maximum(s, ALPHA * s).reshape(1, 1)


def _prologue(feat, W, alv, arv):
    return pl.pallas_call(
        _prologue_body,
        out_shape=[
            jax.ShapeDtypeStruct((NPAD, D), jnp.float32),
            jax.ShapeDtypeStruct((NPAD, 1), jnp.float32),
            jax.ShapeDtypeStruct((NPAD, 1), jnp.float32),
            jax.ShapeDtypeStruct((1, 1), jnp.float32),
        ],
    )(feat, W, alv, arv)


def _sc_mesh():
    return plsc.VectorSubcoreMesh(core_axis_name="c", subcore_axis_name="s",
                                  num_cores=NC, num_subcores=NS)


def _sc_compiler_params():
    cp = pltpu.CompilerParams()
    if "needs_layout_passes" in pltpu.CompilerParams.__dataclass_fields__:
        cp = dataclasses.replace(cp, needs_layout_passes=False)
    return cp


def _weights_body(src_hbm, dst_hbm, el_hbm, er_hbm, m_hbm,
                  ee_hbm, pden_hbm,
                  el_v, er_v, src_v, dst_v, ee_v, m_v, den_v):
    cid = lax.axis_index("c")
    sid = lax.axis_index("s")
    wid = sid * NC + cid

    pltpu.sync_copy(el_hbm, el_v)
    pltpu.sync_copy(er_hbm, er_v)
    pltpu.sync_copy(m_hbm, m_v)
    mvec = m_v[pl.ds(0, LANES)]

    zero16 = jnp.zeros((LANES,), jnp.float32)

    @pl.loop(0, NPAD // D)
    def _(r):
        @pl.loop(0, D // LANES)
        def _(c):
            den_v[r, pl.ds(c * LANES, LANES)] = zero16

    for q in range(NCHUNK):
        wq = wid * NCHUNK + q
        pltpu.sync_copy(src_hbm.at[wq], src_v)
        pltpu.sync_copy(dst_hbm.at[wq], dst_v)

        @pl.loop(0, CHK)
        def _(j):
            for k in range(BLK // LANES):
                sv = src_v[j, pl.ds(k * LANES, LANES)]
                dv = dst_v[j, pl.ds(k * LANES, LANES)]
                s = plsc.load_gather(el_v, [sv]) + plsc.load_gather(er_v, [dv])
                e = jnp.maximum(s, ALPHA * s)
                ee = jnp.exp(e - mvec)
                ee_v[j, pl.ds(k * LANES, LANES)] = ee
                plsc.addupdate_scatter(den_v, [dv >> 7, dv & 127], ee)

        pltpu.sync_copy(ee_v, ee_hbm.at[wq])

    pltpu.sync_copy(den_v, pden_hbm.at[wid])


def _sc_weights(srcb, dstb, el, er, m128):
    f = pl.kernel(
        _weights_body,
        out_type=[
            jax.ShapeDtypeStruct((NW * NCHUNK, CHK, BLK), jnp.float32),
            jax.ShapeDtypeStruct((NW, NPAD // D, D), jnp.float32),
        ],
        mesh=_sc_mesh(),
        scratch_types=[
            pltpu.VMEM((NPAD,), jnp.float32),
            pltpu.VMEM((NPAD,), jnp.float32),
            pltpu.VMEM((CHK, BLK), jnp.int32),
            pltpu.VMEM((CHK, BLK), jnp.int32),
            pltpu.VMEM((CHK, BLK), jnp.float32),
            pltpu.VMEM((BLK,), jnp.float32),
            pltpu.VMEM((NPAD // D, D), jnp.float32),
        ],
        compiler_params=_sc_compiler_params(),
    )
    return f(srcb, dstb, el, er, m128)


def _scale_block(rows_v, ee_v, j):
    jv = jnp.zeros((LANES,), jnp.int32) + j

    @pl.loop(0, BLK, unroll=8)
    def _(r):
        rv = jnp.zeros((LANES,), jnp.int32) + r
        wvec = plsc.load_gather(ee_v, [jv, rv])
        for c in range(D // LANES):
            sl = pl.ds(c * LANES, LANES)
            rows_v[r, sl] = rows_v[r, sl] * wvec


def _agg_body(ft_hbm, src_hbm, dst_hbm, ee_hbm, prow_hbm,
              acc_rows, src_v, dst_v, ee_v, rows0, rows1, gsem0, gsem1,
              ssem0, ssem1):
    cid = lax.axis_index("c")
    sid = lax.axis_index("s")
    wid = sid * NC + cid

    zero16 = jnp.zeros((LANES,), jnp.float32)

    @pl.loop(0, BLK)
    def _(r):
        @pl.loop(0, D // LANES)
        def _(c):
            rows0[r, pl.ds(c * LANES, LANES)] = zero16

    base = sid * RPT
    for i in range(RPT // BLK):
        pltpu.sync_copy(rows0, acc_rows.at[pl.ds(base + i * BLK, BLK)])

    plsc.subcore_barrier()

    @pl.loop(0, NCHUNK)
    def _(q):
        wq = wid * NCHUNK + q
        pltpu.sync_copy(src_hbm.at[wq], src_v)
        pltpu.sync_copy(dst_hbm.at[wq], dst_v)
        pltpu.sync_copy(ee_hbm.at[wq], ee_v)

        pltpu.async_copy(ft_hbm.at[src_v.at[0]], rows0, gsem0)
        pltpu.async_copy(ft_hbm.at[src_v.at[1]], rows1, gsem1)

        @pl.loop(0, CHK // 2)
        def _(jj):
            j0 = jj * 2
            for off, rows, gsem, ssem in ((0, rows0, gsem0, ssem0),
                                          (1, rows1, gsem1, ssem1)):
                j = j0 + off
                pltpu.make_async_copy(ft_hbm.at[src_v.at[j]], rows, gsem).wait()
                _scale_block(rows, ee_v, j)
                pltpu.async_copy(rows, acc_rows.at[dst_v.at[j]], ssem,
                                 add=True)
            for off, rows, gsem, ssem in ((0, rows0, gsem0, ssem0),
                                          (1, rows1, gsem1, ssem1)):
                j = j0 + off
                pltpu.make_async_copy(rows, acc_rows.at[dst_v.at[j]],
                                      ssem).wait()

                @pl.when(j + 2 < CHK)
                def _():
                    pltpu.async_copy(ft_hbm.at[src_v.at[j + 2]], rows, gsem)

    plsc.subcore_barrier()

    pltpu.sync_copy(acc_rows.at[pl.ds(base, RPT)],
                    prow_hbm.at[pl.ds(cid * NPAD + base, RPT)])


def _sc_aggregate(ft, srcb, dstb, eeb):
    f = pl.kernel(
        _agg_body,
        out_type=jax.ShapeDtypeStruct((NC * NPAD, D), jnp.float32),
        mesh=_sc_mesh(),
        scratch_types=[
            pltpu.VMEM_SHARED((NPAD, D), jnp.float32),
            pltpu.VMEM((CHK, BLK), jnp.int32),
            pltpu.VMEM((CHK, BLK), jnp.int32),
            pltpu.VMEM((CHK, BLK), jnp.float32),
            pltpu.VMEM((BLK, D), jnp.float32),
            pltpu.VMEM((BLK, D), jnp.float32),
            pltpu.SemaphoreType.DMA,
            pltpu.SemaphoreType.DMA,
            pltpu.SemaphoreType.DMA,
            pltpu.SemaphoreType.DMA,
        ],
        compiler_params=_sc_compiler_params(),
    )
    return f(ft, srcb, dstb, eeb)


def _epilogue_body(prow_ref, pden_ref, out_ref):
    ps = prow_ref[:N] + prow_ref[NPAD:NPAD + N]
    den = jnp.sum(pden_ref[:, :N], axis=0)[:, None]
    out_ref[...] = ps / (den + 1e-9)


def _epilogue(prow, pden):
    return pl.pallas_call(
        _epilogue_body,
        out_shape=jax.ShapeDtypeStruct((N, D), jnp.float32),
    )(prow, pden)


def kernel(feat, edge_index, W, attn_l, attn_r):
    alv = attn_l.reshape(1, D)
    arv = attn_r.reshape(1, D)
    ft, el, er, m = _prologue(feat, W, alv, arv)
    el = el.reshape(NPAD)
    er = er.reshape(NPAD)
    m128 = jnp.broadcast_to(m.reshape(()), (BLK,))
    pad = N + jnp.arange(EPAD - E, dtype=jnp.int32) % (NPAD - N)
    srcb = jnp.concatenate([edge_index[0], pad]).reshape(NW * NCHUNK, CHK, BLK)
    dstb = jnp.concatenate([edge_index[1], pad]).reshape(NW * NCHUNK, CHK, BLK)
    ee, pden = _sc_weights(srcb, dstb, el, er, m128)
    pden = pden.reshape(NW, NPAD)
    prow = _sc_aggregate(ft, srcb, dstb, ee)
    out = _epilogue(prow, pden)
    return out.reshape(N, 1, D)


# speedup vs baseline: 1.0853x; 1.0853x over previous
import dataclasses

import jax
import jax.numpy as jnp
from jax import lax
from jax.experimental import pallas as pl
from jax.experimental.pallas import tpu as pltpu
from jax.experimental.pallas import tpu_sc as plsc

N = 10000
E = 320000
IN_DIM = 128
D = 128
ALPHA = 0.2

NC = 2
NS = 16
NW = NC * NS
BLK = 128
CHK = 16
NCHUNK = 5
EPW = NCHUNK * CHK * BLK
EPAD = NW * EPW
NPAD = 10240
RPT = NPAD // NS
DLN = 16
LANES = 16
NEG = -1e30


def _prologue_body(feat_ref, w_ref, al_ref, ar_ref, ft_ref, el_ref, er_ref,
                   m_ref):
    ft = jnp.dot(feat_ref[...], w_ref[...], preferred_element_type=jnp.float32)
    ft_ref[:N] = ft
    ft_ref[N:] = jnp.zeros((NPAD - N, D), jnp.float32)
    el = jnp.sum(ft * al_ref[...], axis=1, keepdims=True)
    er = jnp.sum(ft * ar_ref[...], axis=1, keepdims=True)
    el_ref[:N] = el
    el_ref[N:] = jnp.full((NPAD - N, 1), NEG, jnp.float32)
    er_ref[:N] = er
    er_ref[N:] = jnp.full((NPAD - N, 1), NEG, jnp.float32)
    s = jnp.max(el) + jnp.max(er)
    m_ref[...] = jnp.maximum(s, ALPHA * s).reshape(1, 1)


def _prologue(feat, W, alv, arv):
    return pl.pallas_call(
        _prologue_body,
        out_shape=[
            jax.ShapeDtypeStruct((NPAD, D), jnp.float32),
            jax.ShapeDtypeStruct((NPAD, 1), jnp.float32),
            jax.ShapeDtypeStruct((NPAD, 1), jnp.float32),
            jax.ShapeDtypeStruct((1, 1), jnp.float32),
        ],
    )(feat, W, alv, arv)


def _sc_mesh():
    return plsc.VectorSubcoreMesh(core_axis_name="c", subcore_axis_name="s",
                                  num_cores=NC, num_subcores=NS)


def _sc_compiler_params():
    cp = pltpu.CompilerParams()
    if "needs_layout_passes" in pltpu.CompilerParams.__dataclass_fields__:
        cp = dataclasses.replace(cp, needs_layout_passes=False)
    return cp


def _weights_body(src_hbm, dst_hbm, el_hbm, er_hbm, m_hbm,
                  ee_hbm, pden_hbm,
                  el_v, er_v, src_v, dst_v, ee_v, m_v, den_v):
    cid = lax.axis_index("c")
    sid = lax.axis_index("s")
    wid = sid * NC + cid

    pltpu.sync_copy(el_hbm, el_v)
    pltpu.sync_copy(er_hbm, er_v)
    pltpu.sync_copy(m_hbm, m_v)
    mvec = m_v[pl.ds(0, LANES)]

    zero16 = jnp.zeros((LANES,), jnp.float32)

    @pl.loop(0, NPAD // D)
    def _(r):
        @pl.loop(0, D // LANES)
        def _(c):
            den_v[r, pl.ds(c * LANES, LANES)] = zero16

    for q in range(NCHUNK):
        wq = wid * NCHUNK + q
        pltpu.sync_copy(src_hbm.at[wq], src_v)
        pltpu.sync_copy(dst_hbm.at[wq], dst_v)

        @pl.loop(0, CHK)
        def _(j):
            for k in range(BLK // LANES):
                sv = src_v[j, pl.ds(k * LANES, LANES)]
                dv = dst_v[j, pl.ds(k * LANES, LANES)]
                s = plsc.load_gather(el_v, [sv]) + plsc.load_gather(er_v, [dv])
                e = jnp.maximum(s, ALPHA * s)
                ee = jnp.exp(e - mvec)
                ee_v[j, pl.ds(k * LANES, LANES)] = ee
                plsc.addupdate_scatter(den_v, [dv >> 7, dv & 127], ee)

        pltpu.sync_copy(ee_v, ee_hbm.at[wq])

    pltpu.sync_copy(den_v, pden_hbm.at[wid])


def _sc_weights(srcb, dstb, el, er, m128):
    f = pl.kernel(
        _weights_body,
        out_type=[
            jax.ShapeDtypeStruct((NW * NCHUNK, CHK, BLK), jnp.float32),
            jax.ShapeDtypeStruct((NW, NPAD // D, D), jnp.float32),
        ],
        mesh=_sc_mesh(),
        scratch_types=[
            pltpu.VMEM((NPAD,), jnp.float32),
            pltpu.VMEM((NPAD,), jnp.float32),
            pltpu.VMEM((CHK, BLK), jnp.int32),
            pltpu.VMEM((CHK, BLK), jnp.int32),
            pltpu.VMEM((CHK, BLK), jnp.float32),
            pltpu.VMEM((BLK,), jnp.float32),
            pltpu.VMEM((NPAD // D, D), jnp.float32),
        ],
        compiler_params=_sc_compiler_params(),
    )
    return f(srcb, dstb, el, er, m128)


def _scale_block(rows_v, ee_v, j):
    @pl.loop(0, BLK // LANES, unroll=2)
    def _(g):
        eerow = ee_v[j, pl.ds(g * LANES, LANES)]
        for r in range(LANES):
            iv = jnp.full((LANES, 1), r, jnp.int32)
            wvec = lax.gather(
                eerow, iv,
                lax.GatherDimensionNumbers(offset_dims=(),
                                           collapsed_slice_dims=(0,),
                                           start_index_map=(0,)),
                (1,), mode=lax.GatherScatterMode.PROMISE_IN_BOUNDS)
            row = g * LANES + r
            for c in range(D // LANES):
                sl = pl.ds(c * LANES, LANES)
                rows_v[row, sl] = rows_v[row, sl] * wvec


def _agg_body(ft_hbm, src_hbm, dst_hbm, ee_hbm, prow_hbm,
              acc_rows, src_v, dst_v, ee_v, rows0, rows1, gsem0, gsem1,
              ssem0, ssem1):
    cid = lax.axis_index("c")
    sid = lax.axis_index("s")
    wid = sid * NC + cid

    zero16 = jnp.zeros((LANES,), jnp.float32)

    @pl.loop(0, BLK)
    def _(r):
        @pl.loop(0, D // LANES)
        def _(c):
            rows0[r, pl.ds(c * LANES, LANES)] = zero16

    base = sid * RPT
    for i in range(RPT // BLK):
        pltpu.sync_copy(rows0, acc_rows.at[pl.ds(base + i * BLK, BLK)])

    plsc.subcore_barrier()

    @pl.loop(0, NCHUNK)
    def _(q):
        wq = wid * NCHUNK + q
        pltpu.sync_copy(src_hbm.at[wq], src_v)
        pltpu.sync_copy(dst_hbm.at[wq], dst_v)
        pltpu.sync_copy(ee_hbm.at[wq], ee_v)

        pltpu.async_copy(ft_hbm.at[src_v.at[0]], rows0, gsem0)
        pltpu.async_copy(ft_hbm.at[src_v.at[1]], rows1, gsem1)

        @pl.loop(0, CHK // 2)
        def _(jj):
            j0 = jj * 2
            for off, rows, gsem, ssem in ((0, rows0, gsem0, ssem0),
                                          (1, rows1, gsem1, ssem1)):
                j = j0 + off
                pltpu.make_async_copy(ft_hbm.at[src_v.at[j]], rows, gsem).wait()
                _scale_block(rows, ee_v, j)
                pltpu.async_copy(rows, acc_rows.at[dst_v.at[j]], ssem,
                                 add=True)
            for off, rows, gsem, ssem in ((0, rows0, gsem0, ssem0),
                                          (1, rows1, gsem1, ssem1)):
                j = j0 + off
                pltpu.make_async_copy(rows, acc_rows.at[dst_v.at[j]],
                                      ssem).wait()

                @pl.when(j + 2 < CHK)
                def _():
                    pltpu.async_copy(ft_hbm.at[src_v.at[j + 2]], rows, gsem)

    plsc.subcore_barrier()

    pltpu.sync_copy(acc_rows.at[pl.ds(base, RPT)],
                    prow_hbm.at[pl.ds(cid * NPAD + base, RPT)])


def _sc_aggregate(ft, srcb, dstb, eeb):
    f = pl.kernel(
        _agg_body,
        out_type=jax.ShapeDtypeStruct((NC * NPAD, D), jnp.float32),
        mesh=_sc_mesh(),
        scratch_types=[
            pltpu.VMEM_SHARED((NPAD, D), jnp.float32),
            pltpu.VMEM((CHK, BLK), jnp.int32),
            pltpu.VMEM((CHK, BLK), jnp.int32),
            pltpu.VMEM((CHK, BLK), jnp.float32),
            pltpu.VMEM((BLK, D), jnp.float32),
            pltpu.VMEM((BLK, D), jnp.float32),
            pltpu.SemaphoreType.DMA,
            pltpu.SemaphoreType.DMA,
            pltpu.SemaphoreType.DMA,
            pltpu.SemaphoreType.DMA,
        ],
        compiler_params=_sc_compiler_params(),
    )
    return f(ft, srcb, dstb, eeb)


def _epilogue_body(prow_ref, pden_ref, out_ref):
    ps = prow_ref[:N] + prow_ref[NPAD:NPAD + N]
    den = jnp.sum(pden_ref[:, :N], axis=0)[:, None]
    out_ref[...] = ps / (den + 1e-9)


def _epilogue(prow, pden):
    return pl.pallas_call(
        _epilogue_body,
        out_shape=jax.ShapeDtypeStruct((N, D), jnp.float32),
    )(prow, pden)


def kernel(feat, edge_index, W, attn_l, attn_r):
    alv = attn_l.reshape(1, D)
    arv = attn_r.reshape(1, D)
    ft, el, er, m = _prologue(feat, W, alv, arv)
    el = el.reshape(NPAD)
    er = er.reshape(NPAD)
    m128 = jnp.broadcast_to(m.reshape(()), (BLK,))
    pad = N + jnp.arange(EPAD - E, dtype=jnp.int32) % (NPAD - N)
    srcb = jnp.concatenate([edge_index[0], pad]).reshape(NW * NCHUNK, CHK, BLK)
    dstb = jnp.concatenate([edge_index[1], pad]).reshape(NW * NCHUNK, CHK, BLK)
    ee, pden = _sc_weights(srcb, dstb, el, er, m128)
    pden = pden.reshape(NW, NPAD)
    prow = _sc_aggregate(ft, srcb, dstb, ee)
    out = _epilogue(prow, pden)
    return out.reshape(N, 1, D)
